# baseline (speedup 1.0000x reference)
import jax
import jax.numpy as jnp
from jax import lax
from jax.experimental import pallas as pl
from jax.experimental.pallas import tpu as pltpu

_NSRC = 32
_NB = 8
_W = 2048


def _body(sx_ref, sy_ref, x_ref, y_ref, out_ref, rows, bulk_sem, in_sem,
          out_sem):
    bulk = [
        pltpu.make_async_copy(y_ref.at[b], out_ref.at[b], bulk_sem)
        for b in range(_NB)
    ]
    for c in bulk:
        c.start()

    stage = [
        pltpu.make_async_copy(
            y_ref.at[:, pl.ds(sx_ref[i], 1), :], rows.at[i], in_sem)
        for i in range(_NSRC)
    ]
    for c in stage:
        c.start()
    for c in stage:
        c.wait()

    col = lax.broadcasted_iota(jnp.int32, (1, _W), 1)
    for i in range(_NSRC):
        sy = sy_ref[i]
        for b in range(_NB):
            rows[i, b] += jnp.where(col == sy, x_ref[b, i], 0.0)

    for c in bulk:
        c.wait()

    unstage = [
        pltpu.make_async_copy(
            rows.at[i], out_ref.at[:, pl.ds(sx_ref[i], 1), :], out_sem)
        for i in range(_NSRC)
    ]
    for c in unstage:
        c.start()
    for c in unstage:
        c.wait()


def kernel(Y, X, src_x, src_y):
    return pl.pallas_call(
        _body,
        in_specs=[
            pl.BlockSpec(memory_space=pltpu.SMEM),
            pl.BlockSpec(memory_space=pltpu.SMEM),
            pl.BlockSpec(memory_space=pltpu.SMEM),
            pl.BlockSpec(memory_space=pltpu.MemorySpace.HBM),
        ],
        out_specs=pl.BlockSpec(memory_space=pltpu.MemorySpace.HBM),
        out_shape=jax.ShapeDtypeStruct(Y.shape, Y.dtype),
        scratch_shapes=[
            pltpu.VMEM((_NSRC, _NB, 1, _W), jnp.float32),
            pltpu.SemaphoreType.DMA,
            pltpu.SemaphoreType.DMA,
            pltpu.SemaphoreType.DMA,
        ],
    )(src_x, src_y, X, Y)


# speedup vs baseline: 48.8663x; 48.8663x over previous
import jax
import jax.numpy as jnp
from jax import lax
from jax.experimental import pallas as pl
from jax.experimental.pallas import tpu as pltpu

_R = 1024
_NSRC = 32


def _body(src_x_ref, src_y_ref, x_ref, y_ref, out_ref):
    b = pl.program_id(0)
    rb = pl.program_id(1)
    r0 = rb * _R
    out_ref[...] = y_ref[...]
    col = lax.broadcasted_iota(jnp.int32, (1, 2048), 1)
    for i in range(_NSRC):
        sx = src_x_ref[i]
        sy = src_y_ref[i]

        @pl.when(jnp.logical_and(sx >= r0, sx < r0 + _R))
        def _():
            xl = sx - r0
            xv = x_ref[b, i]
            row = out_ref[0, pl.ds(xl, 1), :]
            out_ref[0, pl.ds(xl, 1), :] = row + jnp.where(col == sy, xv, 0.0)


def kernel(Y, X, src_x, src_y):
    B, H, W = Y.shape
    grid = (B, H // _R)
    return pl.pallas_call(
        _body,
        grid=grid,
        in_specs=[
            pl.BlockSpec(memory_space=pltpu.SMEM),
            pl.BlockSpec(memory_space=pltpu.SMEM),
            pl.BlockSpec(memory_space=pltpu.SMEM),
            pl.BlockSpec((1, _R, W), lambda b, r: (b, r, 0)),
        ],
        out_specs=pl.BlockSpec((1, _R, W), lambda b, r: (b, r, 0)),
        out_shape=jax.ShapeDtypeStruct(Y.shape, Y.dtype),
        compiler_params=pltpu.CompilerParams(
            dimension_semantics=("parallel", "parallel"),
        ),
    )(src_x, src_y, X, Y)
